# Initial kernel scaffold; baseline (speedup 1.0000x reference)
import jax
import jax.numpy as jnp
from jax.experimental import pallas as pl


def kernel(x, W1, b1, W2, b2, We, be):
    raise NotImplementedError("write your pallas kernel here")


# speedup vs baseline: 2.2912x; 2.2912x over previous
import functools
import jax
import jax.numpy as jnp
from jax.experimental import pallas as pl
from jax.experimental.pallas import tpu as pltpu

B, D, H, E, K, F = 4096, 1024, 128, 8, 2, 1024


def _router_body(x_ref, W1_ref, b1_ref, W2_ref, b2_ref, cw_ref):
    x = x_ref[...]
    h1 = jnp.tanh(
        jnp.dot(x, W1_ref[...], preferred_element_type=jnp.float32) + b1_ref[...]
    )
    logits = (
        jnp.dot(h1, W2_ref[...], preferred_element_type=jnp.float32) + b2_ref[...]
    )
    m = jnp.max(logits, axis=1, keepdims=True)
    ex = jnp.exp(logits - m)
    l = ex / jnp.sum(ex, axis=1, keepdims=True)
    iota = jax.lax.broadcasted_iota(jnp.int32, l.shape, 1)
    m1 = jnp.max(l, axis=1, keepdims=True)
    a1 = jnp.min(jnp.where(l == m1, iota, E), axis=1, keepdims=True)
    l2 = jnp.where(iota == a1, -1.0, l)
    m2 = jnp.max(l2, axis=1, keepdims=True)
    a2 = jnp.min(jnp.where(l2 == m2, iota, E), axis=1, keepdims=True)
    t = jnp.exp(m2 - m1)
    s1 = 1.0 / (1.0 + t)
    s2 = t / (1.0 + t)
    cw_ref[...] = jnp.where(iota == a1, s1, 0.0) + jnp.where(iota == a2, s2, 0.0)


def _expert_body(cw_ref, x_ref, We_ref, be_ref, out_ref):
    e = pl.program_id(1)
    y = (
        jnp.dot(x_ref[...], We_ref[0], preferred_element_type=jnp.float32)
        + be_ref[0]
    )
    iota = jax.lax.broadcasted_iota(jnp.int32, cw_ref.shape, 1)
    w = jnp.sum(jnp.where(iota == e, cw_ref[...], 0.0), axis=1, keepdims=True)
    contrib = w * y

    @pl.when(e == 0)
    def _():
        out_ref[...] = contrib

    @pl.when(e > 0)
    def _():
        out_ref[...] += contrib


def kernel(x, W1, b1, W2, b2, We, be):
    nb = 8
    bb = B // nb
    cw = pl.pallas_call(
        _router_body,
        grid=(nb,),
        in_specs=[
            pl.BlockSpec((bb, D), lambda i: (i, 0)),
            pl.BlockSpec((D, H), lambda i: (0, 0)),
            pl.BlockSpec((1, H), lambda i: (0, 0)),
            pl.BlockSpec((H, E), lambda i: (0, 0)),
            pl.BlockSpec((1, E), lambda i: (0, 0)),
        ],
        out_specs=pl.BlockSpec((bb, E), lambda i: (i, 0)),
        out_shape=jax.ShapeDtypeStruct((B, E), jnp.float32),
    )(x, W1, b1.reshape(1, H), W2, b2.reshape(1, E))

    nf = 2
    fb = F // nf
    out = pl.pallas_call(
        _expert_body,
        grid=(nf, E),
        in_specs=[
            pl.BlockSpec((B, E), lambda f, e: (0, 0)),
            pl.BlockSpec((B, D), lambda f, e: (0, 0)),
            pl.BlockSpec((1, D, fb), lambda f, e: (e, 0, f)),
            pl.BlockSpec((1, 1, fb), lambda f, e: (e, 0, f)),
        ],
        out_specs=pl.BlockSpec((B, fb), lambda f, e: (0, f)),
        out_shape=jax.ShapeDtypeStruct((B, F), jnp.float32),
    )(cw, x, We, be.reshape(E, 1, F))
    return out
